# Initial kernel scaffold; baseline (speedup 1.0000x reference)
import jax
import jax.numpy as jnp
from jax.experimental import pallas as pl


def kernel(edge_fea, hyperedge_index, W1, b1, W2, b2, g1, beta1, g2, beta2, a):
    raise NotImplementedError("write your pallas kernel here")


# speedup vs baseline: 6.5025x; 6.5025x over previous
import functools

import jax
import jax.numpy as jnp
from jax import lax
from jax.experimental import pallas as pl
from jax.experimental.pallas import tpu as pltpu
from jax.experimental.pallas import tpu_sc as plsc

N_NODES = 10000
N_HE = 10000
N_INC = 320000
D = 128
EPS = 1e-5

NC, NS, L = 2, 16, 16
NW = NC * NS
CHUNK = 80
PER_W = N_INC // NW
NCH = PER_W // CHUNK
ACC_PAD = 10240
ROWS_PER_S = ACC_PAD // NS
DUMP_K = ROWS_PER_S // CHUNK
DEG_PAD = 10240
DEG_PER_S = DEG_PAD // NS
DEG_CH = (N_INC // NS) // CHUNK

_mesh = plsc.VectorSubcoreMesh(core_axis_name="c", subcore_axis_name="s")


@functools.partial(
    pl.kernel,
    out_type=jax.ShapeDtypeStruct((NC, NS * DUMP_K, CHUNK, D), jnp.float32),
    mesh=_mesh,
    scratch_types=[
        pltpu.VMEM_SHARED((ACC_PAD, D), jnp.float32),
        pltpu.VMEM((NCH, CHUNK), jnp.int32),
        pltpu.VMEM((NCH, CHUNK), jnp.int32),
        pltpu.VMEM((CHUNK, D), jnp.float32),
        pltpu.SemaphoreType.DMA,
    ],
)
def _segsum(src, gidx, sidx, out, acc, gbuf, sbuf, rows, sem):
    c = lax.axis_index("c")
    s = lax.axis_index("s")
    wid = c * NS + s

    def _zrow(i, carry):
        for j in range(D // L):
            rows[i, pl.ds(j * L, L)] = jnp.zeros((L,), jnp.float32)
        return carry

    lax.fori_loop(0, CHUNK, _zrow, 0)
    for k in range(DUMP_K):
        pltpu.sync_copy(rows, acc.at[pl.ds(s * ROWS_PER_S + k * CHUNK, CHUNK)])
    plsc.subcore_barrier()

    pltpu.sync_copy(gidx.at[wid], gbuf)
    pltpu.sync_copy(sidx.at[wid], sbuf)

    def _body(j, carry):
        pltpu.async_copy(src.at[gbuf.at[j]], rows, sem).wait()
        pltpu.sync_copy(rows, acc.at[sbuf.at[j]], add=True)
        return carry

    lax.fori_loop(0, NCH, _body, 0)
    plsc.subcore_barrier()

    for k in range(DUMP_K):
        pltpu.sync_copy(acc.at[pl.ds(s * ROWS_PER_S + k * CHUNK, CHUNK)], rows)
        pltpu.sync_copy(rows, out.at[c, s * DUMP_K + k])


@functools.partial(
    pl.kernel,
    out_type=(jax.ShapeDtypeStruct((DEG_PAD,), jnp.float32),
              jax.ShapeDtypeStruct((DEG_PAD,), jnp.float32)),
    mesh=_mesh,
    scratch_types=[
        pltpu.VMEM_SHARED((DEG_PAD,), jnp.float32),
        pltpu.VMEM((DEG_CH, CHUNK), jnp.int32),
        pltpu.VMEM((CHUNK,), jnp.float32),
        pltpu.VMEM((DEG_PER_S,), jnp.float32),
    ],
)
def _degrees(he_idx, nd_idx, binv_out, dinv_out, dacc, islab, ones, lbuf):
    c = lax.axis_index("c")
    s = lax.axis_index("s")

    for k in range(CHUNK // L):
        ones[pl.ds(k * L, L)] = jnp.ones((L,), jnp.float32)
    for k in range(DEG_PER_S // L):
        lbuf[pl.ds(k * L, L)] = jnp.zeros((L,), jnp.float32)
    pltpu.sync_copy(lbuf, dacc.at[pl.ds(s * DEG_PER_S, DEG_PER_S)])
    plsc.subcore_barrier()

    def _hist(idx_src, out_ref):
        pltpu.sync_copy(idx_src.at[s], islab)

        def _body(j, carry):
            pltpu.sync_copy(ones, dacc.at[islab.at[j]], add=True)
            return carry

        lax.fori_loop(0, DEG_CH, _body, 0)
        plsc.subcore_barrier()
        pltpu.sync_copy(dacc.at[pl.ds(s * DEG_PER_S, DEG_PER_S)], lbuf)
        for k in range(DEG_PER_S // L):
            v = lbuf[pl.ds(k * L, L)]
            lbuf[pl.ds(k * L, L)] = jnp.where(v > 0, 1.0 / v, jnp.zeros((L,), jnp.float32))
        pltpu.sync_copy(lbuf, out_ref.at[pl.ds(s * DEG_PER_S, DEG_PER_S)])

    @pl.when(c == 0)
    def _():
        _hist(he_idx, binv_out)

    @pl.when(c == 1)
    def _():
        _hist(nd_idx, dinv_out)


BLK = 2000


def _k1_body(x_ref, w1t_ref, b1_ref, w2t_ref, out_ref, w2st_ref):
    out_ref[...] = jnp.dot(x_ref[...], w1t_ref[...],
                           preferred_element_type=jnp.float32) + b1_ref[...]

    @pl.when(pl.program_id(0) == 0)
    def _():
        w2t = w2t_ref[...]
        mean = jnp.mean(w2t, axis=0, keepdims=True)
        var = jnp.sum((w2t - mean) ** 2, axis=0, keepdims=True) / (D - 1)
        w2st_ref[...] = (w2t - mean) * lax.rsqrt(var + 1e-5)


def _k1(x, w1t, b1, w2t):
    return pl.pallas_call(
        _k1_body,
        grid=(N_NODES // BLK,),
        in_specs=[
            pl.BlockSpec((BLK, D), lambda i: (i, 0)),
            pl.BlockSpec((D, D), lambda i: (0, 0)),
            pl.BlockSpec((1, D), lambda i: (0, 0)),
            pl.BlockSpec((D, D), lambda i: (0, 0)),
        ],
        out_specs=[
            pl.BlockSpec((BLK, D), lambda i: (i, 0)),
            pl.BlockSpec((D, D), lambda i: (0, 0)),
        ],
        out_shape=[
            jax.ShapeDtypeStruct((N_NODES, D), jnp.float32),
            jax.ShapeDtypeStruct((D, D), jnp.float32),
        ],
    )(x, w1t, b1, w2t)


def _comb_body(p0_ref, p1_ref, r_ref, out_ref):
    out_ref[...] = (p0_ref[...] + p1_ref[...]) * r_ref[...]


def _comb(p0, p1, r):
    return pl.pallas_call(
        _comb_body,
        grid=(N_NODES // BLK,),
        in_specs=[
            pl.BlockSpec((BLK, D), lambda i: (i, 0)),
            pl.BlockSpec((BLK, D), lambda i: (i, 0)),
            pl.BlockSpec((BLK, 1), lambda i: (i, 0)),
        ],
        out_specs=pl.BlockSpec((BLK, D), lambda i: (i, 0)),
        out_shape=jax.ShapeDtypeStruct((N_NODES, D), jnp.float32),
    )(p0, p1, r)


def _ln_prelu(x, g_ref, beta_ref, a_ref):
    mu = jnp.mean(x, axis=-1, keepdims=True)
    var = jnp.mean((x - mu) ** 2, axis=-1, keepdims=True)
    xn = (x - mu) * lax.rsqrt(var + EPS) * g_ref[...] + beta_ref[...]
    return jnp.where(xn >= 0, xn, a_ref[...] * xn)


def _k2_body(q0_ref, q1_ref, r_ref, g_ref, beta_ref, a_ref, w2st_ref, b2_ref,
             out_ref):
    x = (q0_ref[...] + q1_ref[...]) * r_ref[...]
    h = _ln_prelu(x, g_ref, beta_ref, a_ref)
    out_ref[...] = jnp.dot(h, w2st_ref[...],
                           preferred_element_type=jnp.float32) + b2_ref[...]


def _k2(q0, q1, r, g, beta, a, w2st, b2):
    return pl.pallas_call(
        _k2_body,
        grid=(N_NODES // BLK,),
        in_specs=[
            pl.BlockSpec((BLK, D), lambda i: (i, 0)),
            pl.BlockSpec((BLK, D), lambda i: (i, 0)),
            pl.BlockSpec((BLK, 1), lambda i: (i, 0)),
            pl.BlockSpec((1, D), lambda i: (0, 0)),
            pl.BlockSpec((1, D), lambda i: (0, 0)),
            pl.BlockSpec((1, 1), lambda i: (0, 0)),
            pl.BlockSpec((D, D), lambda i: (0, 0)),
            pl.BlockSpec((1, D), lambda i: (0, 0)),
        ],
        out_specs=pl.BlockSpec((BLK, D), lambda i: (i, 0)),
        out_shape=jax.ShapeDtypeStruct((N_NODES, D), jnp.float32),
    )(q0, q1, r, g, beta, a, w2st, b2)


def _k3_body(q0_ref, q1_ref, r_ref, g_ref, beta_ref, a_ref, out_ref):
    x = (q0_ref[...] + q1_ref[...]) * r_ref[...]
    out_ref[...] = _ln_prelu(x, g_ref, beta_ref, a_ref)


def _k3(q0, q1, r, g, beta, a):
    return pl.pallas_call(
        _k3_body,
        grid=(N_NODES // BLK,),
        in_specs=[
            pl.BlockSpec((BLK, D), lambda i: (i, 0)),
            pl.BlockSpec((BLK, D), lambda i: (i, 0)),
            pl.BlockSpec((BLK, 1), lambda i: (i, 0)),
            pl.BlockSpec((1, D), lambda i: (0, 0)),
            pl.BlockSpec((1, D), lambda i: (0, 0)),
            pl.BlockSpec((1, 1), lambda i: (0, 0)),
        ],
        out_specs=pl.BlockSpec((BLK, D), lambda i: (i, 0)),
        out_shape=jax.ShapeDtypeStruct((N_NODES, D), jnp.float32),
    )(q0, q1, r, g, beta, a)


def kernel(edge_fea, hyperedge_index, W1, b1, W2, b2, g1, beta1, g2, beta2, a):
    node_idx = hyperedge_index[0]
    he_idx = hyperedge_index[1]

    gidx = node_idx.reshape(NW, NCH, CHUNK)
    sidx = he_idx.reshape(NW, NCH, CHUNK)
    he_idx_h = he_idx.reshape(NS, DEG_CH, CHUNK)
    nd_idx_h = node_idx.reshape(NS, DEG_CH, CHUNK)

    binv_p, dinv_p = _degrees(he_idx_h, nd_idx_h)
    binv = binv_p[:N_HE].reshape(N_HE, 1)
    dinv = dinv_p[:N_NODES].reshape(N_NODES, 1)

    b1r = b1.reshape(1, D)
    b2r = b2.reshape(1, D)
    g1r, beta1r = g1.reshape(1, D), beta1.reshape(1, D)
    g2r, beta2r = g2.reshape(1, D), beta2.reshape(1, D)
    ar = a.reshape(1, 1)

    xw1, w2st = _k1(edge_fea, W1.T, b1r, W2.T)

    def _seg(src, g, s_):
        return _segsum(src, g, s_).reshape(NC, ACC_PAD, D)[:, :N_NODES]

    pA = _seg(xw1, gidx, sidx)
    he1 = _comb(pA[0], pA[1], binv)
    pB = _seg(he1, sidx, gidx)
    xw2 = _k2(pB[0], pB[1], dinv, g1r, beta1r, ar, w2st, b2r)

    pA2 = _seg(xw2, gidx, sidx)
    he2 = _comb(pA2[0], pA2[1], binv)
    pB2 = _seg(he2, sidx, gidx)
    out = _k3(pB2[0], pB2[1], dinv, g2r, beta2r, ar)
    return out
